# baseline (speedup 1.0000x reference)
import jax
import jax.numpy as jnp
from jax.experimental import pallas as pl
from jax.experimental.pallas import tpu as pltpu


def _mm_mask_kernel(e0_ref, e1_ref, a0_ref, a1_ref, w_ref, out_ref):
    w = w_ref[...]
    for h, (e_ref, a_ref) in enumerate(((e0_ref, a0_ref), (e1_ref, a1_ref))):
        e = e_ref[0]
        logits_t = jax.lax.dot_general(
            w, e,
            dimension_numbers=(((1,), (1,)), ((), ())),
            preferred_element_type=jnp.float32,
        )
        att = a_ref[0]
        out_ref[h] = jnp.where(att, logits_t, -jnp.inf)


def kernel(emb_sentences, att_sentences, W):
    B, L, D = emb_sentences.shape
    NL = W.shape[0]
    att3 = att_sentences.reshape(B, 1, L)

    out = pl.pallas_call(
        _mm_mask_kernel,
        grid=(B // 2,),
        in_specs=[
            pl.BlockSpec((1, L, D), lambda i: (2 * i, 0, 0)),
            pl.BlockSpec((1, L, D), lambda i: (2 * i + 1, 0, 0)),
            pl.BlockSpec((1, 1, L), lambda i: (2 * i, 0, 0)),
            pl.BlockSpec((1, 1, L), lambda i: (2 * i + 1, 0, 0)),
            pl.BlockSpec((NL, D), lambda i: (0, 0)),
        ],
        out_specs=pl.BlockSpec((2, NL, L), lambda i: (i, 0, 0)),
        out_shape=jax.ShapeDtypeStruct((B, NL, L), jnp.float32),
        compiler_params=pltpu.CompilerParams(
            dimension_semantics=("parallel",),
        ),
    )(emb_sentences, emb_sentences, att3, att3, W)
    return out.transpose(0, 2, 1)


# speedup vs baseline: 1.0283x; 1.0283x over previous
import jax
import jax.numpy as jnp
from jax.experimental import pallas as pl
from jax.experimental.pallas import tpu as pltpu


def _mm_mask_kernel(emb_ref, att_ref, w_ref, out_ref):
    e = emb_ref[0]
    w = w_ref[...]
    logits_t = jax.lax.dot_general(
        w, e,
        dimension_numbers=(((1,), (1,)), ((), ())),
        preferred_element_type=jnp.float32,
    )
    att = att_ref[0]
    out_ref[0] = jnp.where(att, logits_t, -jnp.inf)


def kernel(emb_sentences, att_sentences, W):
    B, L, D = emb_sentences.shape
    NL = W.shape[0]
    att3 = att_sentences.reshape(B, 1, L)

    out = pl.pallas_call(
        _mm_mask_kernel,
        grid=(B,),
        in_specs=[
            pl.BlockSpec((1, L, D), lambda i: (i, 0, 0)),
            pl.BlockSpec((1, 1, L), lambda i: (i, 0, 0)),
            pl.BlockSpec((NL, D), lambda i: (0, 0)),
        ],
        out_specs=pl.BlockSpec((1, NL, L), lambda i: (i, 0, 0)),
        out_shape=jax.ShapeDtypeStruct((B, NL, L), jnp.float32),
        compiler_params=pltpu.CompilerParams(
            dimension_semantics=("parallel",),
        ),
    )(emb_sentences, att3, W)
    return out.transpose(0, 2, 1)
